# baseline (speedup 1.0000x reference)
import functools

import jax
import jax.numpy as jnp
from jax.experimental import pallas as pl
from jax.experimental.pallas import tpu as pltpu

_TOP_K = 8


def _router_body(x_ref, w_ref, b_ref, n_ref, rout_ref, idx_ref, ns_ref,
                 *, bm, e, k, c):
    acc = jnp.dot(x_ref[...], w_ref[...],
                  preferred_element_type=jnp.float32,
                  precision=jax.lax.Precision.DEFAULT)
    acc = acc + b_ref[...]
    logits = acc[:, :e]
    nlog = acc[:, e:]
    softplus = jnp.maximum(nlog, 0.0) + jnp.log1p(jnp.exp(-jnp.abs(nlog)))
    ns_ref[...] = logits + n_ref[...] * softplus

    iota_e = jax.lax.broadcasted_iota(jnp.int32, (c, e), 1)
    iota_k = jax.lax.broadcasted_iota(jnp.int32, (c, k), 1)

    def chunk_body(ci, carry):
        off = ci * c
        noisy = ns_ref[pl.ds(off, c), :]
        v = noisy
        sel = jnp.zeros((c, e), jnp.bool_)
        idx_out = jnp.zeros((c, k), jnp.int32)
        m0 = None
        for step in range(k):
            m = jnp.max(v, axis=1, keepdims=True)
            if step == 0:
                m0 = m
            idx = jnp.min(jnp.where(v == m, iota_e, e), axis=1,
                          keepdims=True)
            hit = iota_e == idx
            sel = jnp.logical_or(sel, hit)
            v = jnp.where(hit, -jnp.inf, v)
            idx_out = idx_out + jnp.where(iota_k == step, idx, 0)
        idx_ref[pl.ds(off, c), :] = idx_out
        ex = jnp.where(sel, jnp.exp(noisy - m0), 0.0)
        rout_ref[pl.ds(off, c), :] = ex / jnp.sum(ex, axis=1, keepdims=True)
        return carry

    jax.lax.fori_loop(0, bm // c, chunk_body, 0)


def kernel(x, rng_key, W_logits, b_logits, W_noise, b_noise):
    b, s, d = x.shape
    e = W_logits.shape[1]
    k = _TOP_K
    m = b * s

    raw_noise = jax.random.normal(jax.random.key(rng_key), (b, s, e),
                                  dtype=jnp.float32)
    xm = x.reshape(m, d)
    nm = raw_noise.reshape(m, e)
    wc = jnp.concatenate([W_logits, W_noise], axis=1)
    bc = jnp.concatenate([b_logits, b_noise]).reshape(1, 2 * e)

    bm = 1024
    c = 64
    grid = (m // bm,)

    rout, idx = pl.pallas_call(
        functools.partial(_router_body, bm=bm, e=e, k=k, c=c),
        grid=grid,
        in_specs=[
            pl.BlockSpec((bm, d), lambda i: (i, 0)),
            pl.BlockSpec((d, 2 * e), lambda i: (0, 0)),
            pl.BlockSpec((1, 2 * e), lambda i: (0, 0)),
            pl.BlockSpec((bm, e), lambda i: (i, 0)),
        ],
        out_specs=[
            pl.BlockSpec((bm, e), lambda i: (i, 0)),
            pl.BlockSpec((bm, k), lambda i: (i, 0)),
        ],
        out_shape=[
            jax.ShapeDtypeStruct((m, e), jnp.float32),
            jax.ShapeDtypeStruct((m, k), jnp.int32),
        ],
        scratch_shapes=[pltpu.VMEM((bm, e), jnp.float32)],
    )(xm, wc, bc, nm)

    return rout.reshape(b, s, e), idx.reshape(b, s, k)


# speedup vs baseline: 3.5464x; 3.5464x over previous
import functools

import jax
import jax.numpy as jnp
from jax.experimental import pallas as pl
from jax.experimental.pallas import tpu as pltpu

_TOP_K = 8
_R0 = (13, 15, 26, 6)
_R1 = (17, 29, 16, 24)


def _rotl(x, r):
    return (x << jnp.uint32(r)) | (x >> jnp.uint32(32 - r))


def _tf_rounds(x0, x1, rots):
    for r in rots:
        x0 = x0 + x1
        x1 = x0 ^ _rotl(x1, r)
    return x0, x1


def _noise_bits(base, k1, k2, bm, e):
    iota_r = jax.lax.broadcasted_iota(jnp.int32, (bm, e), 0)
    iota_c = jax.lax.broadcasted_iota(jnp.int32, (bm, e), 1)
    g = (base + iota_r * e + iota_c).astype(jnp.uint32)
    ks2 = jnp.uint32(0x1BD11BDA) ^ k1 ^ k2
    x0 = jnp.full((bm, e), k1, jnp.uint32)
    x1 = g + k2
    x0, x1 = _tf_rounds(x0, x1, _R0)
    x0, x1 = x0 + k2, x1 + ks2 + jnp.uint32(1)
    x0, x1 = _tf_rounds(x0, x1, _R1)
    x0, x1 = x0 + ks2, x1 + k1 + jnp.uint32(2)
    x0, x1 = _tf_rounds(x0, x1, _R0)
    x0, x1 = x0 + k1, x1 + k2 + jnp.uint32(3)
    x0, x1 = _tf_rounds(x0, x1, _R1)
    x0, x1 = x0 + k2, x1 + ks2 + jnp.uint32(4)
    x0, x1 = _tf_rounds(x0, x1, _R0)
    x0, x1 = x0 + ks2, x1 + k1 + jnp.uint32(5)
    return x0 ^ x1


def _router_body(kd_ref, x_ref, w_ref, b_ref, rout_ref, idx_ref,
                 *, bm, e, k):
    i = pl.program_id(0)
    bits = _noise_bits(i * (bm * e), kd_ref[0], kd_ref[1], bm, e)
    fb = (bits >> jnp.uint32(9)) | jnp.uint32(0x3F800000)
    fl = jax.lax.bitcast_convert_type(fb, jnp.float32) - 1.0
    lo = jnp.float32(-0.99999994)
    u = jnp.maximum(lo, fl * 2.0 + lo)
    raw_noise = jnp.float32(1.4142135) * jax.lax.erf_inv(u)

    acc = jnp.dot(x_ref[...], w_ref[...],
                  preferred_element_type=jnp.float32,
                  precision=jax.lax.Precision.DEFAULT)
    acc = acc + b_ref[...]
    logits = acc[:, :e]
    nlog = acc[:, e:]
    softplus = jnp.maximum(nlog, 0.0) + jnp.log1p(jnp.exp(-jnp.abs(nlog)))
    noisy = logits + raw_noise * softplus

    iota_e = jax.lax.broadcasted_iota(jnp.int32, (bm, e), 1)
    iota_k = jax.lax.broadcasted_iota(jnp.int32, (bm, k), 1)
    v = noisy
    sel = jnp.zeros((bm, e), jnp.bool_)
    idx_out = jnp.zeros((bm, k), jnp.int32)
    m0 = None
    for step in range(k):
        m = jnp.max(v, axis=1, keepdims=True)
        if step == 0:
            m0 = m
        idx = jnp.min(jnp.where(v == m, iota_e, e), axis=1, keepdims=True)
        hit = iota_e == idx
        sel = jnp.logical_or(sel, hit)
        v = jnp.where(hit, -jnp.inf, v)
        idx_out = idx_out + jnp.where(iota_k == step, idx, 0)

    idx_ref[...] = idx_out
    ex = jnp.where(sel, jnp.exp(noisy - m0), 0.0)
    rout_ref[...] = ex / jnp.sum(ex, axis=1, keepdims=True)


def kernel(x, rng_key, W_logits, b_logits, W_noise, b_noise):
    b, s, d = x.shape
    e = W_logits.shape[1]
    k = _TOP_K
    m = b * s

    kd = jax.random.key_data(jax.random.key(rng_key)).astype(jnp.uint32)
    xm = x.reshape(m, d)
    wc = jnp.concatenate([W_logits, W_noise], axis=1)
    bc = jnp.concatenate([b_logits, b_noise]).reshape(1, 2 * e)

    bm = 1024
    grid = (m // bm,)

    rout, idx = pl.pallas_call(
        functools.partial(_router_body, bm=bm, e=e, k=k),
        grid=grid,
        in_specs=[
            pl.BlockSpec(memory_space=pltpu.SMEM),
            pl.BlockSpec((bm, d), lambda i: (i, 0)),
            pl.BlockSpec((d, 2 * e), lambda i: (0, 0)),
            pl.BlockSpec((1, 2 * e), lambda i: (0, 0)),
        ],
        out_specs=[
            pl.BlockSpec((bm, e), lambda i: (i, 0)),
            pl.BlockSpec((bm, k), lambda i: (i, 0)),
        ],
        out_shape=[
            jax.ShapeDtypeStruct((m, e), jnp.float32),
            jax.ShapeDtypeStruct((m, k), jnp.int32),
        ],
    )(kd, xm, wc, bc)

    return rout.reshape(b, s, e), idx.reshape(b, s, k)


# speedup vs baseline: 5.8490x; 1.6493x over previous
import functools

import jax
import jax.numpy as jnp
from jax.experimental import pallas as pl
from jax.experimental.pallas import tpu as pltpu

_TOP_K = 8
_R0 = (13, 15, 26, 6)
_R1 = (17, 29, 16, 24)


def _rotl(x, r):
    return (x << jnp.uint32(r)) | (x >> jnp.uint32(32 - r))


def _tf_rounds(x0, x1, rots):
    for r in rots:
        x0 = x0 + x1
        x1 = x0 ^ _rotl(x1, r)
    return x0, x1


def _noise_bits(g, k1, k2):
    ks2 = jnp.uint32(0x1BD11BDA) ^ k1 ^ k2
    x0 = jnp.full(g.shape, k1, jnp.uint32)
    x1 = g + k2
    x0, x1 = _tf_rounds(x0, x1, _R0)
    x0, x1 = x0 + k2, x1 + ks2 + jnp.uint32(1)
    x0, x1 = _tf_rounds(x0, x1, _R1)
    x0, x1 = x0 + ks2, x1 + k1 + jnp.uint32(2)
    x0, x1 = _tf_rounds(x0, x1, _R0)
    x0, x1 = x0 + k1, x1 + k2 + jnp.uint32(3)
    x0, x1 = _tf_rounds(x0, x1, _R1)
    x0, x1 = x0 + k2, x1 + ks2 + jnp.uint32(4)
    x0, x1 = _tf_rounds(x0, x1, _R0)
    x0, x1 = x0 + ks2, x1 + k1 + jnp.uint32(5)
    return x0 ^ x1


def _router_body(kd_ref, x_ref, w_ref, b_ref, rout_ref, idx_ref,
                 *, bt, e, k):
    i = pl.program_id(0)
    iota_r = jax.lax.broadcasted_iota(jnp.int32, (e, bt), 0)
    iota_c = jax.lax.broadcasted_iota(jnp.int32, (e, bt), 1)
    g = (i * (bt * e) + iota_c * e + iota_r).astype(jnp.uint32)
    bits = _noise_bits(g, kd_ref[0], kd_ref[1])
    fb = (bits >> jnp.uint32(9)) | jnp.uint32(0x3F800000)
    fl = jax.lax.bitcast_convert_type(fb, jnp.float32) - 1.0
    lo = jnp.float32(-0.99999994)
    u = jnp.maximum(lo, fl * 2.0 + lo)
    raw_noise = jnp.float32(1.4142135) * jax.lax.erf_inv(u)

    acc = jax.lax.dot_general(
        w_ref[...], x_ref[...],
        dimension_numbers=(((1,), (1,)), ((), ())),
        preferred_element_type=jnp.float32,
        precision=jax.lax.Precision.DEFAULT)
    acc = acc + b_ref[...]
    logits = acc[:e, :]
    nlog = acc[e:, :]
    softplus = jnp.maximum(nlog, 0.0) + jnp.log1p(jnp.exp(-jnp.abs(nlog)))
    noisy = logits + raw_noise * softplus

    iota_k0 = jax.lax.broadcasted_iota(jnp.int32, (k, bt), 0)
    v = noisy
    sel = jnp.zeros((e, bt), jnp.bool_)
    idx_out = jnp.zeros((k, bt), jnp.int32)
    m0 = None
    for step in range(k):
        m = jnp.max(v, axis=0, keepdims=True)
        if step == 0:
            m0 = m
        idx = jnp.min(jnp.where(v == m, iota_r, e), axis=0, keepdims=True)
        hit = iota_r == idx
        sel = jnp.logical_or(sel, hit)
        v = jnp.where(hit, -jnp.inf, v)
        idx_out = idx_out + jnp.where(iota_k0 == step, idx, 0)

    idx_ref[...] = idx_out
    ex = jnp.where(sel, jnp.exp(noisy - m0), 0.0)
    rout_ref[...] = ex / jnp.sum(ex, axis=0, keepdims=True)


def kernel(x, rng_key, W_logits, b_logits, W_noise, b_noise):
    b, s, d = x.shape
    e = W_logits.shape[1]
    k = _TOP_K
    m = b * s

    kd = jax.random.key_data(jax.random.key(rng_key)).astype(jnp.uint32)
    xm = x.reshape(m, d)
    wct = jnp.concatenate([W_logits, W_noise], axis=1).T
    bct = jnp.concatenate([b_logits, b_noise]).reshape(2 * e, 1)

    bt = 1024
    grid = (m // bt,)

    rout_t, idx_t = pl.pallas_call(
        functools.partial(_router_body, bt=bt, e=e, k=k),
        grid=grid,
        in_specs=[
            pl.BlockSpec(memory_space=pltpu.SMEM),
            pl.BlockSpec((bt, d), lambda i: (i, 0)),
            pl.BlockSpec((2 * e, d), lambda i: (0, 0)),
            pl.BlockSpec((2 * e, 1), lambda i: (0, 0)),
        ],
        out_specs=[
            pl.BlockSpec((e, bt), lambda i: (0, i)),
            pl.BlockSpec((k, bt), lambda i: (0, i)),
        ],
        out_shape=[
            jax.ShapeDtypeStruct((e, m), jnp.float32),
            jax.ShapeDtypeStruct((k, m), jnp.int32),
        ],
    )(kd, xm, wct, bct)

    return (rout_t.T.reshape(b, s, e), idx_t.T.reshape(b, s, k))
